# baseline (speedup 1.0000x reference)
import jax
import jax.numpy as jnp
from jax.experimental import pallas as pl
from jax.experimental.pallas import tpu as pltpu

_N = 8192
_IN = 256
_C = 4
_ROWS = _N // 1024
_RPC = _ROWS // _C
_CHUNK = _N // _C
_K = 10


def _weldon_kernel(x_ref, w_ref, b_ref, o_ref, s_ref):
    c = pl.program_id(1)
    w = w_ref[...]
    rows = []
    for r in range(_RPC):
        xc = x_ref[0, pl.ds(r * 1024, 1024), :]
        rows.append(
            jax.lax.dot_general(
                w, xc, (((1,), (1,)), ((), ())),
                preferred_element_type=jnp.float32,
            )
        )
    s_ref[c] = jnp.concatenate(rows, axis=0)

    @pl.when(c == _C - 1)
    def _finish():
        s = s_ref[...]
        ninf = jnp.float32(float("-inf"))
        pinf = jnp.float32(float("inf"))
        t_acc = jnp.zeros((1, 1, 1), jnp.float32)
        b_acc = jnp.zeros((1, 1, 1), jnp.float32)
        t_need = jnp.full((1, 1, 1), _K, jnp.float32)
        b_need = jnp.full((1, 1, 1), _K, jnp.float32)
        t_cur = s
        b_cur = s
        for _ in range(_K):
            tm = jnp.max(t_cur, axis=(0, 1, 2), keepdims=True)
            bm = jnp.min(b_cur, axis=(0, 1, 2), keepdims=True)
            t_eq = t_cur == tm
            b_eq = b_cur == bm
            t_cnt = jnp.sum(jnp.where(t_eq, 1.0, 0.0), axis=(0, 1, 2),
                            keepdims=True)
            b_cnt = jnp.sum(jnp.where(b_eq, 1.0, 0.0), axis=(0, 1, 2),
                            keepdims=True)
            t_take = jnp.minimum(t_cnt, t_need)
            b_take = jnp.minimum(b_cnt, b_need)
            t_acc = t_acc + jnp.where(t_take > 0, tm * t_take, 0.0)
            b_acc = b_acc + jnp.where(b_take > 0, bm * b_take, 0.0)
            t_need = t_need - t_take
            b_need = b_need - b_take
            t_cur = jnp.where(t_eq, ninf, t_cur)
            b_cur = jnp.where(b_eq, pinf, b_cur)

        mean = (t_acc + b_acc) / jnp.float32(2 * _K) + b_ref[...].reshape(1, 1, 1)
        o_ref[...] = jax.nn.sigmoid(mean)


@jax.jit
def kernel(x, W, b):
    B = x.shape[0]
    out = pl.pallas_call(
        _weldon_kernel,
        grid=(B, _C),
        in_specs=[
            pl.BlockSpec((1, _CHUNK, _IN), lambda i, j: (i, j, 0)),
            pl.BlockSpec((1, _IN), lambda i, j: (0, 0)),
            pl.BlockSpec((1, 1), lambda i, j: (0, 0)),
        ],
        out_specs=pl.BlockSpec((1, 1, 1), lambda i, j: (i, 0, 0)),
        out_shape=jax.ShapeDtypeStruct((B, 1, 1), jnp.float32),
        scratch_shapes=[pltpu.VMEM((_C, _RPC, 1024), jnp.float32)],
    )(x, W, jnp.reshape(b, (1, 1)))
    return out.reshape(-1)


# speedup vs baseline: 1.8303x; 1.8303x over previous
import jax
import jax.numpy as jnp
from jax.experimental import pallas as pl
from jax.experimental.pallas import tpu as pltpu

_N = 8192
_IN = 256
_CHUNKS = 8
_CHUNK = _N // _CHUNKS
_K = 10


def _weldon_kernel(x_ref, w_ref, b_ref, o_ref):
    w = w_ref[...]
    rows = []
    for c in range(_CHUNKS):
        xc = x_ref[0, pl.ds(c * _CHUNK, _CHUNK), :]
        rows.append(
            jax.lax.dot_general(
                w, xc, (((1,), (1,)), ((), ())),
                preferred_element_type=jnp.float32,
            )
        )
    s = jnp.concatenate(rows, axis=0)

    ninf = jnp.float32(float("-inf"))
    pinf = jnp.float32(float("inf"))
    t_acc = jnp.zeros((1, 1), jnp.float32)
    b_acc = jnp.zeros((1, 1), jnp.float32)
    t_need = jnp.full((1, 1), _K, jnp.float32)
    b_need = jnp.full((1, 1), _K, jnp.float32)
    t_cur = s
    b_cur = s
    for _ in range(_K):
        tm = jnp.max(t_cur, axis=(0, 1), keepdims=True)
        bm = jnp.min(b_cur, axis=(0, 1), keepdims=True)
        t_eq = t_cur == tm
        b_eq = b_cur == bm
        t_cnt = jnp.sum(jnp.where(t_eq, 1.0, 0.0), axis=(0, 1), keepdims=True)
        b_cnt = jnp.sum(jnp.where(b_eq, 1.0, 0.0), axis=(0, 1), keepdims=True)
        t_take = jnp.minimum(t_cnt, t_need)
        b_take = jnp.minimum(b_cnt, b_need)
        t_acc = t_acc + jnp.where(t_take > 0, tm * t_take, 0.0)
        b_acc = b_acc + jnp.where(b_take > 0, bm * b_take, 0.0)
        t_need = t_need - t_take
        b_need = b_need - b_take
        t_cur = jnp.where(t_eq, ninf, t_cur)
        b_cur = jnp.where(b_eq, pinf, b_cur)

    mean = (t_acc + b_acc) / jnp.float32(2 * _K) + b_ref[...]
    o_ref[0] = jax.nn.sigmoid(mean)


@jax.jit
def kernel(x, W, b):
    B = x.shape[0]
    out = pl.pallas_call(
        _weldon_kernel,
        grid=(B,),
        in_specs=[
            pl.BlockSpec((1, _N, _IN), lambda i: (i, 0, 0)),
            pl.BlockSpec((1, _IN), lambda i: (0, 0)),
            pl.BlockSpec((1, 1), lambda i: (0, 0)),
        ],
        out_specs=pl.BlockSpec((1, 1, 1), lambda i: (i, 0, 0)),
        out_shape=jax.ShapeDtypeStruct((B, 1, 1), jnp.float32),
        compiler_params=pltpu.CompilerParams(
            dimension_semantics=("parallel",),
        ),
    )(x, W, jnp.reshape(b, (1, 1)))
    return out.reshape(-1)
